# baseline (speedup 1.0000x reference)
import jax
import jax.numpy as jnp
from jax.experimental import pallas as pl


def _body(x_ref, ce_ref, pos_ref, out_ref):
    _, T, hs, C, W = x_ref.shape
    D = ce_ref.shape[1]
    h = pl.program_id(1)
    x = x_ref[0]
    z = jnp.swapaxes(x, 2, 3)
    L = jnp.transpose(z, (1, 2, 0, 3))
    L = L.reshape(hs * W * T, C)
    y = jnp.dot(L, ce_ref[...], preferred_element_type=jnp.float32)
    p = pos_ref[pl.ds(h * hs * W, hs * W), :]
    y = y.reshape(hs * W, T, D) + p[:, None, :]
    out_ref[0] = y.reshape(hs, W, T, D)


@jax.jit
def kernel(inputs, channel_embeddings, positional_embeddings):
    B, T, H, W, C = inputs.shape
    _, D = channel_embeddings.shape
    HW = H * W
    hs = 64
    nh = H // hs

    x = jnp.swapaxes(inputs, 3, 4)
    pos = positional_embeddings.reshape(HW, D)

    out = pl.pallas_call(
        _body,
        grid=(B, nh),
        in_specs=[
            pl.BlockSpec((1, T, hs, C, W), lambda b, h: (b, 0, h, 0, 0)),
            pl.BlockSpec((C, D), lambda b, h: (0, 0)),
            pl.BlockSpec((HW, D), lambda b, h: (0, 0)),
        ],
        out_specs=pl.BlockSpec((1, hs, W, T, D), lambda b, h: (b, h, 0, 0, 0)),
        out_shape=jax.ShapeDtypeStruct((B, H, W, T, D), jnp.float32),
    )(x, channel_embeddings, pos)

    return out
